# baseline (speedup 1.0000x reference)
import functools

import jax
import jax.numpy as jnp
from jax import lax
from jax.experimental import pallas as pl
from jax.experimental.pallas import tpu as pltpu
from jax.experimental.pallas import tpu_sc as plsc

VOCAB = 100000
D = 128
NQ = 64
ROW_BLK = 2000

NW = 32
CHUNK = 128
FIRE = 2
N_CHUNKS = 5


def _vocab_mlp_kernel(emb_ref, w1_ref, b1_ref, w2_ref, b2_ref, p_ref):
    h = jnp.tanh(
        jnp.dot(emb_ref[...], w1_ref[...], preferred_element_type=jnp.float32)
        + b1_ref[...]
    )
    p_ref[...] = (
        jnp.dot(h, w2_ref[...], preferred_element_type=jnp.float32)
        + b2_ref[...]
    )


def _vocab_mlp(emb, W1, b1, W2, b2):
    grid = VOCAB // ROW_BLK
    return pl.pallas_call(
        _vocab_mlp_kernel,
        grid=(grid,),
        in_specs=[
            pl.BlockSpec((ROW_BLK, D), lambda i: (i, 0)),
            pl.BlockSpec((D, D), lambda i: (0, 0)),
            pl.BlockSpec((1, D), lambda i: (0, 0)),
            pl.BlockSpec((D, NQ), lambda i: (0, 0)),
            pl.BlockSpec((1, NQ), lambda i: (0, 0)),
        ],
        out_specs=pl.BlockSpec((ROW_BLK, NQ), lambda i: (i, 0)),
        out_shape=jax.ShapeDtypeStruct((VOCAB, NQ), jnp.float32),
    )(emb, W1, b1.reshape(1, D), W2, b2.reshape(1, NQ))


def _make_sc_gather(n_tokens):
    per_w = n_tokens // NW
    n_steps = per_w // (FIRE * CHUNK)
    idx_rows = per_w // CHUNK

    mesh = plsc.VectorSubcoreMesh(core_axis_name="c", subcore_axis_name="s")
    info = plsc.get_sparse_core_info()
    nc = info.num_cores

    step_rows = FIRE * CHUNK
    assert n_steps % 2 == 0 and n_steps >= 4

    @functools.partial(
        pl.kernel,
        out_type=jax.ShapeDtypeStruct((n_tokens, NQ), jnp.float32),
        mesh=mesh,
        scratch_types=[
            pltpu.VMEM((idx_rows, CHUNK), jnp.int32),
            pltpu.VMEM((2, step_rows, NQ), jnp.float32),
            pltpu.SemaphoreType.DMA,
            pltpu.SemaphoreType.DMA,
            pltpu.SemaphoreType.DMA,
        ],
        compiler_params=pltpu.CompilerParams(use_tc_tiling_on_sc=False),
    )
    def gather_kernel(table_hbm, idx_hbm, out_hbm, idx_v, rows_v, sem_g,
                      sem_w0, sem_w1):
        wid = lax.axis_index("s") * nc + lax.axis_index("c")
        base = wid * per_w
        sem_w = (sem_w0, sem_w1)
        pltpu.sync_copy(idx_hbm.at[pl.ds(wid * idx_rows, idx_rows)], idx_v)

        def fire_and_wait(step, b):
            copies = []
            for f in range(FIRE):
                copies.append(
                    pltpu.async_copy(
                        table_hbm.at[idx_v.at[step * FIRE + f]],
                        rows_v.at[b].at[pl.ds(f * CHUNK, CHUNK)],
                        sem_g,
                    )
                )
            for c in copies:
                c.wait()

        def writeback(step, b):
            pltpu.async_copy(
                rows_v.at[b],
                out_hbm.at[pl.ds(base + step * step_rows, step_rows)],
                sem_w[b],
            )

        def drain(b):
            pltpu.make_async_copy(
                rows_v.at[b],
                out_hbm.at[pl.ds(base, step_rows)],
                sem_w[b],
            ).wait()

        for b in (0, 1):
            fire_and_wait(b, b)
            writeback(b, b)

        def step2(g2, carry):
            for b in (0, 1):
                step = g2 * 2 + b
                drain(b)
                fire_and_wait(step, b)
                writeback(step, b)
            return carry

        lax.fori_loop(1, n_steps // 2, step2, 0)
        drain(0)
        drain(1)

    return gather_kernel


def kernel(tok, emb, W1, b1, W2, b2):
    B, L = tok.shape
    table = _vocab_mlp(emb, W1, b1, W2, b2)
    lc = L // N_CHUNKS
    chunks = []
    sc_gather = _make_sc_gather(B * lc)
    for i in range(N_CHUNKS):
        idx = tok[:, i * lc:(i + 1) * lc].T.reshape(-1, CHUNK)
        g = sc_gather(table, idx.astype(jnp.int32))
        chunks.append(jnp.transpose(g.reshape(lc, B, NQ), (0, 2, 1)))
    t = jnp.concatenate(chunks, axis=0)
    return jnp.transpose(t, (2, 0, 1))
